# baseline (speedup 1.0000x reference)
import functools

import jax
import jax.numpy as jnp
from jax import lax
from jax.experimental import pallas as pl
from jax.experimental.pallas import tpu as pltpu, tpu_sc as plsc

NUM_CLASSES = 10
NC, NS, L = 2, 16, 16
NW = NC * NS
BATCH = 16384
ROWS_PER_W = BATCH // NW
GROUPS = ROWS_PER_W // L
WORDS_PER_W = ROWS_PER_W * NUM_CLASSES

_mesh = plsc.VectorSubcoreMesh(core_axis_name="c", subcore_axis_name="s")


@functools.partial(
    pl.kernel,
    out_type=jax.ShapeDtypeStruct((BATCH,), jnp.int32),
    mesh=_mesh,
    compiler_params=pltpu.CompilerParams(
        needs_layout_passes=False,
        skip_device_barrier=True,
        disable_bounds_checks=True,
        disable_semaphore_checks=True,
    ),
    scratch_types=[
        pltpu.VMEM((WORDS_PER_W,), jnp.float32),
        pltpu.VMEM((L,), jnp.int32),
        pltpu.VMEM((ROWS_PER_W,), jnp.int32),
    ],
)
def _label_map(logits_hbm, map_hbm, out_hbm, buf, map_v, out_v):
    wid = lax.axis_index("s") * NC + lax.axis_index("c")
    pltpu.sync_copy(logits_hbm.at[pl.ds(wid * WORDS_PER_W, WORDS_PER_W)], buf)
    pltpu.sync_copy(map_hbm, map_v)

    lane_off = lax.iota(jnp.int32, L) * NUM_CLASSES

    def body(g, carry):
        idx0 = lane_off + g * (L * NUM_CLASSES)
        best = plsc.load_gather(buf, [idx0])
        besti = jnp.zeros((L,), jnp.int32)
        for c in range(1, NUM_CLASSES):
            v = plsc.load_gather(buf, [idx0 + c])
            gt = v > best
            best = jnp.where(gt, v, best)
            besti = jnp.where(gt, jnp.full((L,), c, jnp.int32), besti)
        out_v[pl.ds(g * L, L)] = plsc.load_gather(map_v, [besti])
        return carry

    lax.fori_loop(0, GROUPS, body, 0)
    pltpu.sync_copy(out_v, out_hbm.at[pl.ds(wid * ROWS_PER_W, ROWS_PER_W)])


def kernel(logits, mapping):
    map16 = jnp.pad(mapping.astype(jnp.int32), (0, L - NUM_CLASSES))
    return _label_map(logits.reshape(-1), map16)


# speedup vs baseline: 1.2275x; 1.2275x over previous
import functools

import jax
import jax.numpy as jnp
from jax import lax
from jax.experimental import pallas as pl
from jax.experimental.pallas import tpu as pltpu, tpu_sc as plsc

NUM_CLASSES = 10
NC, NS, L = 2, 16, 16
NW = NC * NS
BATCH = 16384
ROWS_PER_W = BATCH // NW
GROUPS = ROWS_PER_W // L

_mesh = plsc.VectorSubcoreMesh(core_axis_name="c", subcore_axis_name="s")


@functools.partial(
    pl.kernel,
    out_type=jax.ShapeDtypeStruct((BATCH,), jnp.int32),
    mesh=_mesh,
    compiler_params=pltpu.CompilerParams(
        needs_layout_passes=False,
        skip_device_barrier=True,
        disable_bounds_checks=True,
        disable_semaphore_checks=True,
    ),
    scratch_types=[
        pltpu.VMEM((ROWS_PER_W, NUM_CLASSES), jnp.float32),
        pltpu.VMEM((L,), jnp.int32),
        pltpu.VMEM((ROWS_PER_W,), jnp.int32),
    ],
)
def _label_map(logits_hbm, map_hbm, out_hbm, buf, map_v, out_v):
    wid = lax.axis_index("s") * NC + lax.axis_index("c")
    pltpu.sync_copy(logits_hbm.at[pl.ds(wid * ROWS_PER_W, ROWS_PER_W), :], buf)
    pltpu.sync_copy(map_hbm, map_v.at[pl.ds(0, NUM_CLASSES)])

    lane = lax.iota(jnp.int32, L)
    zero = jnp.zeros((L,), jnp.int32)

    def body(g, carry):
        rows = lane + g * L
        best = plsc.load_gather(buf, [rows, zero])
        besti = zero
        for c in range(1, NUM_CLASSES):
            v = plsc.load_gather(buf, [rows, jnp.full((L,), c, jnp.int32)])
            gt = v > best
            best = jnp.where(gt, v, best)
            besti = jnp.where(gt, jnp.full((L,), c, jnp.int32), besti)
        out_v[pl.ds(g * L, L)] = plsc.load_gather(map_v, [besti])
        return carry

    lax.fori_loop(0, GROUPS, body, 0)
    pltpu.sync_copy(out_v, out_hbm.at[pl.ds(wid * ROWS_PER_W, ROWS_PER_W)])


def kernel(logits, mapping):
    return _label_map(logits, mapping.astype(jnp.int32))
